# baseline (speedup 1.0000x reference)
import functools
import jax
import jax.numpy as jnp
from jax.experimental import pallas as pl

_EPS = 1e-5


def _mm_body(x_ref, w_ref, b_ref, out_ref, *, relu):
    y = jax.lax.dot_general(x_ref[...], w_ref[...], (((1,), (0,)), ((), ())),
                            preferred_element_type=jnp.float32) + b_ref[...]
    if relu:
        y = jnp.maximum(y, 0.0)
    out_ref[...] = y


def _mm(x, w, b, relu=True, tm=1000):
    m, k = x.shape
    cout = w.shape[1]
    fn = pl.pallas_call(
        functools.partial(_mm_body, relu=relu),
        grid=(m // tm,),
        in_specs=[pl.BlockSpec((tm, k), lambda i: (i, 0)),
                  pl.BlockSpec((k, cout), lambda i: (0, 0)),
                  pl.BlockSpec((1, cout), lambda i: (0, 0))],
        out_specs=pl.BlockSpec((tm, cout), lambda i: (i, 0)),
        out_shape=jax.ShapeDtypeStruct((m, cout), jnp.float32),
    )
    return fn(x, w, b.reshape(1, cout))


def _bn(x, g, be):
    mu = jnp.mean(x, axis=0, keepdims=True)
    var = jnp.var(x, axis=0, keepdims=True)
    return (x - mu) / jnp.sqrt(var + _EPS) * g + be


def _mlp_apply(layers, x, pad_to=None):
    for (w, b, g, be) in layers:
        if pad_to is not None:
            x = jnp.pad(x, ((0, 0), (0, pad_to - x.shape[1])))
            w = jnp.pad(w, ((0, pad_to - w.shape[0]), (0, 0)))
            pad_to = None
        x = _bn(_mm(x, w, b, relu=True), g, be)
    return x


def _edge_conv(layers, x, edge_index, n):
    src, dst = edge_index[0], edge_index[1]
    x_i = jnp.take(x, dst, axis=0)
    x_j = jnp.take(x, src, axis=0)
    cat = jnp.concatenate([x_i, x_j - x_i], axis=1)
    m = _mlp_apply(layers, cat)
    out = jax.ops.segment_max(m, dst, num_segments=n)
    return jnp.where(jnp.isfinite(out), out, 0.0)


def _gcu(p, x, tpl_ei, geo_ei, n):
    xt = _edge_conv(p["tpl"], x, tpl_ei, n)
    xg = _edge_conv(p["geo"], x, geo_ei, n)
    return _mlp_apply(p["mlp"], jnp.concatenate([xt, xg], axis=1))


def kernel(skin_input, pos, tpl_edge_index, geo_edge_index, batch, params):
    n = pos.shape[0]
    raw = jnp.concatenate([pos, skin_input], axis=1)
    x0 = _mlp_apply(params["mlp1"], raw, pad_to=128)
    x1 = _gcu(params["gcu1"], x0, tpl_edge_index, geo_edge_index, n)
    xg = _mlp_apply(params["mlp2"], x1)
    xg = jax.ops.segment_max(xg, batch, num_segments=8)
    xg = jnp.where(jnp.isfinite(xg), xg, 0.0)
    x2 = _gcu(params["gcu2"], x1, tpl_edge_index, geo_edge_index, n)
    x3 = _gcu(params["gcu3"], x2, tpl_edge_index, geo_edge_index, n)
    xg_n = jnp.take(xg, batch, axis=0)
    x4 = jnp.concatenate([x3, xg_n], axis=1)
    c = params["cls"]
    h = _bn(_mm(x4, c["W1"], c["b1"], relu=True), c["g1"], c["be1"])
    h = _bn(_mm(h, c["W2"], c["b2"], relu=True), c["g2"], c["be2"])
    w3 = jnp.pad(c["W3"], ((0, 0), (0, 128 - c["W3"].shape[1])))
    b3 = jnp.pad(c["b3"], (0, 128 - c["b3"].shape[0]))
    return _mm(h, w3, b3, relu=False)[:, :5]
